# Initial kernel scaffold; baseline (speedup 1.0000x reference)
import functools

import jax
import jax.numpy as jnp
from jax import lax
from jax.experimental import pallas as pl
from jax.experimental.pallas import tpu as pltpu
from jax.experimental.pallas import tpu_sc as plsc

N = 8192
K = 16
HID = 64
FEAT = 64
TW = 80


BM = 128


def _knn_body(p1_ref, p0t_ref, idx_ref, d2_ref):
    p1 = p1_ref[...]
    p0t = p0t_ref[...]
    dot = jnp.dot(p1, p0t, preferred_element_type=jnp.float32)
    n1 = jnp.sum(p1 * p1, axis=1, keepdims=True)
    n0 = jnp.sum(p0t * p0t, axis=0, keepdims=True)
    d2_ref[...] = n1 + n0 - 2.0 * dot
    iota = lax.broadcasted_iota(jnp.int32, (BM, N), 1)
    cols = []
    for _ in range(K):
        vals = d2_ref[...]
        m = jnp.min(vals, axis=1, keepdims=True)
        sel = jnp.where(vals == m, iota, N)
        ij = jnp.min(sel, axis=1, keepdims=True)
        cols.append(ij)
        d2_ref[...] = jnp.where(iota == ij, jnp.float32(jnp.inf), vals)
    idx_ref[...] = jnp.concatenate(cols, axis=1)


def _knn(p1p, p0tp):
    return pl.pallas_call(
        _knn_body,
        grid=(N // BM,),
        in_specs=[
            pl.BlockSpec((BM, 8), lambda i: (i, 0)),
            pl.BlockSpec((8, N), lambda i: (0, 0)),
        ],
        out_specs=pl.BlockSpec((BM, K), lambda i: (i, 0)),
        out_shape=jax.ShapeDtypeStruct((N, K), jnp.int32),
        scratch_shapes=[pltpu.VMEM((BM, N), jnp.float32)],
    )(p1p, p0tp)


TOT = N * K
CH = 512


def _gather_sc(table, idx_flat):
    info = plsc.get_sparse_core_info()
    nw = info.num_cores * info.num_subcores
    bpw = TOT // nw
    nch = bpw // CH
    mesh = plsc.VectorSubcoreMesh(core_axis_name="c", subcore_axis_name="s")

    @functools.partial(
        pl.kernel,
        mesh=mesh,
        out_type=jax.ShapeDtypeStruct((TOT, TW), jnp.float32),
        scratch_types=[
            pltpu.VMEM((CH,), jnp.int32),
            pltpu.VMEM((CH, TW), jnp.float32),
            pltpu.SemaphoreType.DMA,
        ],
    )
    def k(table_hbm, idx_hbm, out_hbm, idx_v, rows_v, sem):
        wid = lax.axis_index("s") * info.num_cores + lax.axis_index("c")
        base0 = wid * bpw

        def body(i, carry):
            base = base0 + i * CH
            pltpu.sync_copy(idx_hbm.at[pl.ds(base, CH)], idx_v)
            pltpu.async_copy(table_hbm.at[idx_v], rows_v, sem).wait()
            pltpu.sync_copy(rows_v, out_hbm.at[pl.ds(base, CH)])
            return carry

        lax.fori_loop(0, nch, body, 0)

    return k(table, idx_flat)


BM3 = 512


def _mlp_body(g_ref, p1_ref, feat_ref, wbig_ref, w3p_ref, wf_ref,
              wh1f_ref, wh1h_ref, c_ref, out_ref):
    gb = g_ref[...]
    y = jnp.dot(gb, wbig_ref[...], preferred_element_type=jnp.float32)
    p1b = p1_ref[...]
    pcon = jnp.dot(p1b, w3p_ref[...], preferred_element_type=jnp.float32)
    coords = gb[:, 64:67].reshape(BM3, K, 3)
    rela = coords - p1b[:, :3][:, None, :]
    dist = jnp.sqrt(jnp.sum(rela * rela, axis=-1, keepdims=True))
    wdist = c_ref[1:2, :].reshape(1, 1, 192)
    y3 = y.reshape(BM3, K, 192) - pcon[:, None, :] + dist * wdist
    ymax = jnp.max(y3, axis=1) + c_ref[0:1, :]
    featb = feat_ref[...]
    frz = jnp.dot(featb, wf_ref[...], preferred_element_type=jnp.float32)
    gate_r = jax.nn.sigmoid(ymax[:, 0:64] + frz[:, 0:64])
    gate_z = jax.nn.sigmoid(ymax[:, 64:128] + frz[:, 64:128])
    h10 = ymax[:, 128:192]
    h11 = jnp.tanh(
        jnp.dot(featb, wh1f_ref[...], preferred_element_type=jnp.float32)
        + jnp.dot(gate_r * h10, wh1h_ref[...], preferred_element_type=jnp.float32)
        + c_ref[2:3, 0:64])
    out_ref[...] = gate_z * h10 + (1.0 - gate_z) * h11


def _mlp(g, p1p, feat_t, wbig, w3p, wf, wh1f, wh1h, consts):
    return pl.pallas_call(
        _mlp_body,
        grid=(N // BM3,),
        in_specs=[
            pl.BlockSpec((BM3 * K, TW), lambda i: (i, 0)),
            pl.BlockSpec((BM3, 8), lambda i: (i, 0)),
            pl.BlockSpec((BM3, FEAT), lambda i: (i, 0)),
            pl.BlockSpec((TW, 192), lambda i: (0, 0)),
            pl.BlockSpec((8, 192), lambda i: (0, 0)),
            pl.BlockSpec((FEAT, 128), lambda i: (0, 0)),
            pl.BlockSpec((64, 64), lambda i: (0, 0)),
            pl.BlockSpec((64, 64), lambda i: (0, 0)),
            pl.BlockSpec((8, 192), lambda i: (0, 0)),
        ],
        out_specs=pl.BlockSpec((BM3, HID), lambda i: (i, 0)),
        out_shape=jax.ShapeDtypeStruct((N, HID), jnp.float32),
    )(g, p1p, feat_t, wbig, w3p, wf, wh1f, wh1h, consts)


def _pack_weights(W_R, b_R, W_Z, b_Z, W_H0, b_H0, W_H1, b_H1):
    wbig = jnp.zeros((TW, 192), jnp.float32)
    wbig = wbig.at[0:64, 0:64].set(W_R[4:68])
    wbig = wbig.at[64:67, 0:64].set(W_R[0:3])
    wbig = wbig.at[0:64, 64:128].set(W_Z[4:68])
    wbig = wbig.at[64:67, 64:128].set(W_Z[0:3])
    wbig = wbig.at[0:64, 128:192].set(W_H0[4:68])
    wbig = wbig.at[64:67, 128:192].set(W_H0[0:3])
    w3p = jnp.zeros((8, 192), jnp.float32)
    w3p = w3p.at[0:3, 0:64].set(W_R[0:3])
    w3p = w3p.at[0:3, 64:128].set(W_Z[0:3])
    w3p = w3p.at[0:3, 128:192].set(W_H0[0:3])
    wdist = jnp.concatenate([W_R[3], W_Z[3], W_H0[3]], axis=0)
    bcat = jnp.concatenate([b_R, b_Z, b_H0], axis=0)
    consts = jnp.zeros((8, 192), jnp.float32)
    consts = consts.at[0, :].set(bcat)
    consts = consts.at[1, :].set(wdist)
    consts = consts.at[2, 0:64].set(b_H1)
    wf = jnp.concatenate([W_R[68:132], W_Z[68:132]], axis=1)
    wh1f = W_H1[0:64]
    wh1h = W_H1[64:128]
    return wbig, w3p, wf, wh1f, wh1h, consts


def kernel(H0, points0, points1, contents1, motions1,
           W_R, b_R, W_Z, b_Z, W_H0, b_H0, W_H1, b_H1):
    p0t = points0[0]
    p1 = points1[0].T
    p1p = jnp.concatenate([p1, jnp.zeros((N, 5), jnp.float32)], axis=1)
    p0tp = jnp.concatenate([p0t, jnp.zeros((5, N), jnp.float32)], axis=0)

    nn_idx = _knn(p1p, p0tp)

    table = jnp.concatenate(
        [H0[0].T, p0t.T, jnp.zeros((N, TW - HID - 3), jnp.float32)], axis=1)
    g = _gather_sc(table, nn_idx.reshape(-1))

    feat_t = jnp.concatenate([contents1[0], motions1[0]], axis=0).T
    wbig, w3p, wf, wh1f, wh1h, consts = _pack_weights(
        W_R, b_R, W_Z, b_Z, W_H0, b_H0, W_H1, b_H1)
    out = _mlp(g, p1p, feat_t, wbig, w3p, wf, wh1f, wh1h, consts)
    return out.T[None]


# speedup vs baseline: 15.6160x; 15.6160x over previous
import functools

import jax
import jax.numpy as jnp
from jax import lax
from jax.experimental import pallas as pl
from jax.experimental.pallas import tpu as pltpu
from jax.experimental.pallas import tpu_sc as plsc

N = 8192
K = 16
HID = 64
FEAT = 64
TW = 128


BM = 128


def _knn_body(p1_ref, p0t_ref, idx_ref, d2_ref):
    p1 = p1_ref[...]
    p0t = p0t_ref[...]
    dot = jnp.dot(p1, p0t, preferred_element_type=jnp.float32)
    n1 = jnp.sum(p1 * p1, axis=1, keepdims=True)
    n0 = jnp.sum(p0t * p0t, axis=0, keepdims=True)
    d2_ref[...] = n1 + n0 - 2.0 * dot
    iota = lax.broadcasted_iota(jnp.int32, (BM, N), 1)
    cols = []
    for _ in range(K):
        vals = d2_ref[...]
        m = jnp.min(vals, axis=1, keepdims=True)
        sel = jnp.where(vals == m, iota, N)
        ij = jnp.min(sel, axis=1, keepdims=True)
        cols.append(ij)
        d2_ref[...] = jnp.where(iota == ij, jnp.float32(jnp.inf), vals)
    idx_ref[...] = jnp.concatenate(cols, axis=1)


def _knn(p1p, p0tp):
    return pl.pallas_call(
        _knn_body,
        grid=(N // BM,),
        in_specs=[
            pl.BlockSpec((BM, 8), lambda i: (i, 0)),
            pl.BlockSpec((8, N), lambda i: (0, 0)),
        ],
        out_specs=pl.BlockSpec((BM, K), lambda i: (i, 0)),
        out_shape=jax.ShapeDtypeStruct((N, K), jnp.int32),
        scratch_shapes=[pltpu.VMEM((BM, N), jnp.float32)],
    )(p1p, p0tp)


TOT = N * K
CH = 512


def _gather_sc(table, idx_flat):
    info = plsc.get_sparse_core_info()
    nw = info.num_cores * info.num_subcores
    bpw = TOT // nw
    nch = bpw // CH
    mesh = plsc.VectorSubcoreMesh(core_axis_name="c", subcore_axis_name="s")

    @functools.partial(
        pl.kernel,
        mesh=mesh,
        out_type=jax.ShapeDtypeStruct((TOT, TW), jnp.float32),
        scratch_types=[
            pltpu.VMEM((CH,), jnp.int32),
            pltpu.VMEM((CH, TW), jnp.float32),
            pltpu.SemaphoreType.DMA,
        ],
    )
    def k(table_hbm, idx_hbm, out_hbm, idx_v, rows_v, sem):
        wid = lax.axis_index("s") * info.num_cores + lax.axis_index("c")
        base0 = wid * bpw

        def body(i, carry):
            base = base0 + i * CH
            pltpu.sync_copy(idx_hbm.at[pl.ds(base, CH)], idx_v)
            pltpu.async_copy(table_hbm.at[idx_v], rows_v, sem).wait()
            pltpu.sync_copy(rows_v, out_hbm.at[pl.ds(base, CH)])
            return carry

        lax.fori_loop(0, nch, body, 0)

    return k(table, idx_flat)


BM3 = 512


def _mlp_body(g_ref, p1_ref, feat_ref, wbig_ref, w3p_ref, wf_ref,
              wh1f_ref, wh1h_ref, c_ref, out_ref):
    gb = g_ref[...]
    y = jnp.dot(gb, wbig_ref[...], preferred_element_type=jnp.float32)
    p1b = p1_ref[...]
    pcon = jnp.dot(p1b, w3p_ref[...], preferred_element_type=jnp.float32)
    coords = gb[:, 64:67].reshape(BM3, K, 3)
    rela = coords - p1b[:, :3][:, None, :]
    dist = jnp.sqrt(jnp.sum(rela * rela, axis=-1, keepdims=True))
    wdist = c_ref[1:2, :].reshape(1, 1, 192)
    y3 = y.reshape(BM3, K, 192) - pcon[:, None, :] + dist * wdist
    ymax = jnp.max(y3, axis=1) + c_ref[0:1, :]
    featb = feat_ref[...]
    frz = jnp.dot(featb, wf_ref[...], preferred_element_type=jnp.float32)
    gate_r = jax.nn.sigmoid(ymax[:, 0:64] + frz[:, 0:64])
    gate_z = jax.nn.sigmoid(ymax[:, 64:128] + frz[:, 64:128])
    h10 = ymax[:, 128:192]
    h11 = jnp.tanh(
        jnp.dot(featb, wh1f_ref[...], preferred_element_type=jnp.float32)
        + jnp.dot(gate_r * h10, wh1h_ref[...], preferred_element_type=jnp.float32)
        + c_ref[2:3, 0:64])
    out_ref[...] = gate_z * h10 + (1.0 - gate_z) * h11


def _mlp(g, p1p, feat_t, wbig, w3p, wf, wh1f, wh1h, consts):
    return pl.pallas_call(
        _mlp_body,
        grid=(N // BM3,),
        in_specs=[
            pl.BlockSpec((BM3 * K, TW), lambda i: (i, 0)),
            pl.BlockSpec((BM3, 8), lambda i: (i, 0)),
            pl.BlockSpec((BM3, FEAT), lambda i: (i, 0)),
            pl.BlockSpec((TW, 192), lambda i: (0, 0)),
            pl.BlockSpec((8, 192), lambda i: (0, 0)),
            pl.BlockSpec((FEAT, 128), lambda i: (0, 0)),
            pl.BlockSpec((64, 64), lambda i: (0, 0)),
            pl.BlockSpec((64, 64), lambda i: (0, 0)),
            pl.BlockSpec((8, 192), lambda i: (0, 0)),
        ],
        out_specs=pl.BlockSpec((BM3, HID), lambda i: (i, 0)),
        out_shape=jax.ShapeDtypeStruct((N, HID), jnp.float32),
    )(g, p1p, feat_t, wbig, w3p, wf, wh1f, wh1h, consts)


def _pack_weights(W_R, b_R, W_Z, b_Z, W_H0, b_H0, W_H1, b_H1):
    wbig = jnp.zeros((TW, 192), jnp.float32)
    wbig = wbig.at[0:64, 0:64].set(W_R[4:68])
    wbig = wbig.at[64:67, 0:64].set(W_R[0:3])
    wbig = wbig.at[0:64, 64:128].set(W_Z[4:68])
    wbig = wbig.at[64:67, 64:128].set(W_Z[0:3])
    wbig = wbig.at[0:64, 128:192].set(W_H0[4:68])
    wbig = wbig.at[64:67, 128:192].set(W_H0[0:3])
    w3p = jnp.zeros((8, 192), jnp.float32)
    w3p = w3p.at[0:3, 0:64].set(W_R[0:3])
    w3p = w3p.at[0:3, 64:128].set(W_Z[0:3])
    w3p = w3p.at[0:3, 128:192].set(W_H0[0:3])
    wdist = jnp.concatenate([W_R[3], W_Z[3], W_H0[3]], axis=0)
    bcat = jnp.concatenate([b_R, b_Z, b_H0], axis=0)
    consts = jnp.zeros((8, 192), jnp.float32)
    consts = consts.at[0, :].set(bcat)
    consts = consts.at[1, :].set(wdist)
    consts = consts.at[2, 0:64].set(b_H1)
    wf = jnp.concatenate([W_R[68:132], W_Z[68:132]], axis=1)
    wh1f = W_H1[0:64]
    wh1h = W_H1[64:128]
    return wbig, w3p, wf, wh1f, wh1h, consts


def kernel(H0, points0, points1, contents1, motions1,
           W_R, b_R, W_Z, b_Z, W_H0, b_H0, W_H1, b_H1):
    p0t = points0[0]
    p1 = points1[0].T
    p1p = jnp.concatenate([p1, jnp.zeros((N, 5), jnp.float32)], axis=1)
    p0tp = jnp.concatenate([p0t, jnp.zeros((5, N), jnp.float32)], axis=0)

    nn_idx = _knn(p1p, p0tp)

    table = jnp.concatenate(
        [H0[0].T, p0t.T, jnp.zeros((N, TW - HID - 3), jnp.float32)], axis=1)
    g = _gather_sc(table, nn_idx.reshape(-1))

    feat_t = jnp.concatenate([contents1[0], motions1[0]], axis=0).T
    wbig, w3p, wf, wh1f, wh1h, consts = _pack_weights(
        W_R, b_R, W_Z, b_Z, W_H0, b_H0, W_H1, b_H1)
    out = _mlp(g, p1p, feat_t, wbig, w3p, wf, wh1f, wh1h, consts)
    return out.T[None]
